# baseline (speedup 1.0000x reference)
import functools

import jax
import jax.numpy as jnp
from jax import lax
from jax.experimental import pallas as pl
from jax.experimental.pallas import tpu as pltpu
from jax.experimental.pallas import tpu_sc as plsc

_L = 16


def _make_sc_kernel(num_nodes, table_rows, n_lambs, n_feat1):
  info = plsc.get_sparse_core_info()
  nc, ns = info.num_cores, info.num_subcores
  nw = nc * ns
  assert num_nodes % nw == 0
  bpw = num_nodes // nw
  n_feat = n_feat1 - 1
  out_w = n_lambs * n_feat
  row_w = n_lambs * n_feat1

  mesh = plsc.VectorSubcoreMesh(core_axis_name="c", subcore_axis_name="s")

  @functools.partial(
      pl.kernel,
      mesh=mesh,
      out_type=jax.ShapeDtypeStruct((num_nodes, out_w), jnp.float32),
      scratch_types=[
          pltpu.VMEM((bpw,), jnp.int32),
          pltpu.VMEM((bpw, row_w), jnp.float32),
          pltpu.VMEM((bpw, out_w), jnp.float32),
          pltpu.SemaphoreType.DMA,
      ],
      compiler_params=pltpu.CompilerParams(use_tc_tiling_on_sc=False),
  )
  def sc_kernel(mem_hbm, nodes_hbm, out_hbm, idx_v, rows_v, out_v, sem):
    wid = lax.axis_index("s") * nc + lax.axis_index("c")
    base = wid * bpw
    pltpu.sync_copy(nodes_hbm.at[pl.ds(base, bpw)], idx_v)
    pltpu.async_copy(mem_hbm.at[idx_v], rows_v, sem).wait()

    def step(i, carry):
      for lamb in range(n_lambs):
        num = rows_v[i, pl.ds(lamb * n_feat1, _L)]
        shifted = rows_v[i, pl.ds(lamb * n_feat1 + 1, _L)]
        den = shifted[_L - 1]
        out_v[i, pl.ds(lamb * n_feat, _L)] = num / den
      return carry

    lax.fori_loop(0, bpw, step, 0, unroll=4)
    pltpu.sync_copy(out_v, out_hbm.at[pl.ds(base, bpw)])

  return sc_kernel


def kernel(memory, nodes):
  num_nodes = nodes.shape[0]
  table_rows, n_lambs, n_feat1 = memory.shape
  sc = _make_sc_kernel(num_nodes, table_rows, n_lambs, n_feat1)
  mem2 = memory.reshape(table_rows, n_lambs * n_feat1)
  return sc(mem2, nodes.astype(jnp.int32))


# speedup vs baseline: 10.7749x; 10.7749x over previous
import functools

import jax
import jax.numpy as jnp
from jax import lax
from jax.experimental import pallas as pl
from jax.experimental.pallas import tpu as pltpu
from jax.experimental.pallas import tpu_sc as plsc

_L = 16
_G = 16
_H = 8
_TB = 128


def _make_sc_kernel(num_nodes, n_lambs, n_feat1, table_rows):
  info = plsc.get_sparse_core_info()
  nc, ns = info.num_cores, info.num_subcores
  nw = nc * ns
  assert num_nodes % (nw * _G) == 0
  bpw = num_nodes // nw
  n_groups = bpw // _G
  n_feat = n_feat1 - 1
  out_w = n_lambs * n_feat

  mesh = plsc.VectorSubcoreMesh(core_axis_name="c", subcore_axis_name="s")

  @functools.partial(
      pl.kernel,
      mesh=mesh,
      out_type=jax.ShapeDtypeStruct((num_nodes, out_w), jnp.float32),
      scratch_types=[
          pltpu.VMEM((bpw,), jnp.int32),
          pltpu.VMEM((_H, n_feat1, n_lambs, _TB), jnp.float32),
          pltpu.VMEM((bpw, out_w), jnp.float32),
          pltpu.SemaphoreType.DMA,
      ],
      compiler_params=pltpu.CompilerParams(needs_layout_passes=False),
  )
  def sc_kernel(mem_hbm, nodes_hbm, out_hbm, idx_v, blk_v, out_v, sem):
    wid = lax.axis_index("s") * nc + lax.axis_index("c")
    base = wid * bpw
    pltpu.sync_copy(nodes_hbm.at[pl.ds(base, bpw)], idx_v)

    lanes = lax.iota(jnp.int32, _L)
    denf = jnp.full((_L,), n_feat, jnp.int32)

    def group(g, carry):
      idx_vec = idx_v[pl.ds(g * _G, _G)]
      for half in range(_G // _H):
        copies = []
        for j in range(_H):
          node = idx_vec[half * _H + j]
          blk = pl.multiple_of((node >> 7) << 7, _TB)
          copies.append(
              pltpu.async_copy(
                  mem_hbm.at[:, :, pl.ds(blk, _TB)], blk_v.at[j], sem
              )
          )
        for c in copies:
          c.wait()
        for j in range(_H):
          jv = jnp.full((_L,), j, jnp.int32)
          rv = jnp.full((_L,), idx_vec[half * _H + j] & 127, jnp.int32)
          for lamb in range(n_lambs):
            lv = jnp.full((_L,), lamb, jnp.int32)
            num = plsc.load_gather(blk_v, [jv, lanes, lv, rv])
            den = plsc.load_gather(blk_v, [jv, denf, lv, rv])
            row = g * _G + half * _H + j
            out_v[row, pl.ds(lamb * n_feat, _L)] = num / den
      return carry

    lax.fori_loop(0, n_groups, group, 0)
    pltpu.sync_copy(out_v, out_hbm.at[pl.ds(base, bpw)])

  return sc_kernel


def kernel(memory, nodes):
  num_nodes = nodes.shape[0]
  table_rows, n_lambs, n_feat1 = memory.shape
  sc = _make_sc_kernel(num_nodes, n_lambs, n_feat1, table_rows)
  mem_t = jnp.transpose(memory, (2, 1, 0))
  return sc(mem_t, nodes.astype(jnp.int32))


# speedup vs baseline: 12.1412x; 1.1268x over previous
import functools

import jax
import jax.numpy as jnp
from jax import lax
from jax.experimental import pallas as pl
from jax.experimental.pallas import tpu as pltpu
from jax.experimental.pallas import tpu_sc as plsc

_L = 16
_B = 4
_TB = 128


def _make_sc_kernel(num_nodes, n_lambs, n_feat1, table_rows):
  info = plsc.get_sparse_core_info()
  nc, ns = info.num_cores, info.num_subcores
  nw = nc * ns
  assert num_nodes % (nw * 2 * _B) == 0
  bpw = num_nodes // nw
  nb = bpw // _B
  n_feat = n_feat1 - 1
  out_w = n_lambs * n_feat

  mesh = plsc.VectorSubcoreMesh(core_axis_name="c", subcore_axis_name="s")

  @functools.partial(
      pl.kernel,
      mesh=mesh,
      out_type=jax.ShapeDtypeStruct((num_nodes, out_w), jnp.float32),
      scratch_types=[
          pltpu.VMEM((bpw + _L,), jnp.int32),
          pltpu.VMEM((2 * _B, n_feat1, n_lambs, _TB), jnp.float32),
          pltpu.VMEM((bpw, out_w), jnp.float32),
          pltpu.SemaphoreType.DMA,
          pltpu.SemaphoreType.DMA,
      ],
      compiler_params=pltpu.CompilerParams(needs_layout_passes=False),
  )
  def sc_kernel(mem_hbm, nodes_hbm, out_hbm, idx_v, blk_v, out_v, s0, s1):
    wid = lax.axis_index("s") * nc + lax.axis_index("c")
    base = wid * bpw
    pltpu.sync_copy(nodes_hbm.at[pl.ds(base, bpw)], idx_v.at[pl.ds(0, bpw)])

    lanes = lax.iota(jnp.int32, _L)
    denf = jnp.full((_L,), n_feat, jnp.int32)

    def fetch(b, ring, sem):
      vec = idx_v[pl.ds(b * _B, _L)]
      for j in range(_B):
        blk = pl.multiple_of((vec[j] >> 7) << 7, _TB)
        pltpu.async_copy(
            mem_hbm.at[:, :, pl.ds(blk, _TB)],
            blk_v.at[ring * _B + j],
            sem,
        )

    def wait_compute(b, ring, sem):
      vec = idx_v[pl.ds(b * _B, _L)]
      for j in range(_B):
        pltpu.make_async_copy(
            mem_hbm.at[:, :, pl.ds(0, _TB)], blk_v.at[ring * _B + j], sem
        ).wait()
      for j in range(_B):
        jv = jnp.full((_L,), ring * _B + j, jnp.int32)
        rv = jnp.full((_L,), vec[j] & (_TB - 1), jnp.int32)
        for lamb in range(n_lambs):
          lv = jnp.full((_L,), lamb, jnp.int32)
          num = plsc.load_gather(blk_v, [jv, lanes, lv, rv])
          den = plsc.load_gather(blk_v, [jv, denf, lv, rv])
          out_v[b * _B + j, pl.ds(lamb * n_feat, _L)] = num / den

    fetch(0, 0, s0)

    def body(i, carry):
      b = i * 2
      fetch(b + 1, 1, s1)
      wait_compute(b, 0, s0)
      fetch(b + 2, 0, s0)
      wait_compute(b + 1, 1, s1)
      return carry

    lax.fori_loop(0, nb // 2 - 1, body, 0)
    fetch(nb - 1, 1, s1)
    wait_compute(nb - 2, 0, s0)
    wait_compute(nb - 1, 1, s1)
    pltpu.sync_copy(out_v, out_hbm.at[pl.ds(base, bpw)])

  return sc_kernel


def kernel(memory, nodes):
  num_nodes = nodes.shape[0]
  table_rows, n_lambs, n_feat1 = memory.shape
  sc = _make_sc_kernel(num_nodes, n_lambs, n_feat1, table_rows)
  mem_t = jnp.transpose(memory, (2, 1, 0))
  return sc(mem_t, nodes.astype(jnp.int32))


# speedup vs baseline: 13.5998x; 1.1201x over previous
import functools

import jax
import jax.numpy as jnp
from jax import lax
from jax.experimental import pallas as pl
from jax.experimental.pallas import tpu as pltpu
from jax.experimental.pallas import tpu_sc as plsc

_L = 16
_B = 8
_TB = 128


def _make_sc_kernel(num_nodes, n_lambs, n_feat1, table_rows):
  info = plsc.get_sparse_core_info()
  nc, ns = info.num_cores, info.num_subcores
  nw = nc * ns
  assert num_nodes % (nw * 2 * _B) == 0
  bpw = num_nodes // nw
  nb = bpw // _B
  n_feat = n_feat1 - 1
  out_w = n_lambs * n_feat

  mesh = plsc.VectorSubcoreMesh(core_axis_name="c", subcore_axis_name="s")

  @functools.partial(
      pl.kernel,
      mesh=mesh,
      out_type=jax.ShapeDtypeStruct((out_w, num_nodes), jnp.float32),
      scratch_types=[
          pltpu.VMEM((bpw + _L,), jnp.int32),
          pltpu.VMEM((2 * _B, n_feat1, n_lambs, _TB), jnp.float32),
          pltpu.VMEM((out_w, bpw), jnp.float32),
          pltpu.SemaphoreType.DMA,
          pltpu.SemaphoreType.DMA,
      ],
      compiler_params=pltpu.CompilerParams(needs_layout_passes=False),
  )
  def sc_kernel(mem_hbm, nodes_hbm, out_hbm, idx_v, blk_v, out_v, s0, s1):
    wid = lax.axis_index("s") * nc + lax.axis_index("c")
    base = wid * bpw
    pltpu.sync_copy(nodes_hbm.at[pl.ds(base, bpw)], idx_v.at[pl.ds(0, bpw)])

    lanes = lax.iota(jnp.int32, _L)
    denf = jnp.full((_L,), n_feat, jnp.int32)

    def fetch(b, ring, sem):
      vec = idx_v[pl.ds(b * _B, _L)]
      for j in range(_B):
        blk = pl.multiple_of((vec[j] >> 7) << 7, _TB)
        pltpu.async_copy(
            mem_hbm.at[:, :, pl.ds(blk, _TB)],
            blk_v.at[ring * _B + j],
            sem,
        )

    def wait_compute(b, ring, sem):
      vec = idx_v[pl.ds(b * _B, _L)]
      for j in range(_B):
        pltpu.make_async_copy(
            mem_hbm.at[:, :, pl.ds(0, _TB)], blk_v.at[ring * _B + j], sem
        ).wait()
      for j in range(_B):
        jv = jnp.full((_L,), ring * _B + j, jnp.int32)
        rv = jnp.full((_L,), vec[j] & (_TB - 1), jnp.int32)
        col = jnp.full((_L,), b * _B + j, jnp.int32)
        for lamb in range(n_lambs):
          lv = jnp.full((_L,), lamb, jnp.int32)
          num = plsc.load_gather(blk_v, [jv, lanes, lv, rv])
          den = plsc.load_gather(blk_v, [jv, denf, lv, rv])
          plsc.store_scatter(
              out_v, [lanes + lamb * n_feat, col], num / den
          )

    fetch(0, 0, s0)

    def body(i, carry):
      b = i * 2
      fetch(b + 1, 1, s1)
      wait_compute(b, 0, s0)
      fetch(b + 2, 0, s0)
      wait_compute(b + 1, 1, s1)
      return carry

    lax.fori_loop(0, nb // 2 - 1, body, 0)
    fetch(nb - 1, 1, s1)
    wait_compute(nb - 2, 0, s0)
    wait_compute(nb - 1, 1, s1)
    pltpu.sync_copy(out_v, out_hbm.at[:, pl.ds(base, bpw)])

  return sc_kernel


def kernel(memory, nodes):
  num_nodes = nodes.shape[0]
  table_rows, n_lambs, n_feat1 = memory.shape
  sc = _make_sc_kernel(num_nodes, n_lambs, n_feat1, table_rows)
  mem_t = jnp.transpose(memory, (2, 1, 0))
  out_t = sc(mem_t, nodes.astype(jnp.int32))
  return out_t.T
